# baseline (speedup 1.0000x reference)
import functools

import jax
import jax.numpy as jnp
from jax import lax
from jax.experimental import pallas as pl
from jax.experimental.pallas import tpu as pltpu
from jax.experimental.pallas import tpu_sc as plsc

N = 10000
N_PAD = 10240
D = 128
NC = 2
NS = 16
L = 16
CH = 64


def _make_spmm(e_pad, local_table, split=None):
    per_tile = e_pad // (NC * NS)
    n_chunks = per_tile // CH
    assert n_chunks * CH == per_tile and n_chunks % 8 == 0
    ca, cb = split if split else (n_chunks, n_chunks)
    assert ca + cb == 2 * n_chunks and ca % 8 == 0 and cb % 8 == 0
    rows_per_tile = N_PAD // NS
    n_wb = rows_per_tile // CH
    n_rb = 2 if local_table else 4
    DEPTH = 1 if local_table else 3

    mesh = plsc.VectorSubcoreMesh(core_axis_name="c", subcore_axis_name="s")

    scratch = [
        pltpu.VMEM((8, 2, CH), jnp.int32),
        pltpu.VMEM((8, CH), jnp.float32),
    ]
    scratch += [pltpu.VMEM((CH, D), jnp.float32)] * n_rb
    if local_table:
        scratch.append(pltpu.VMEM((D, D), jnp.float32))
    scratch.append(pltpu.VMEM_SHARED((N_PAD, D), jnp.float32))
    scratch += [pltpu.SemaphoreType.DMA] * (8 + 2 * n_rb)

    @functools.partial(
        pl.kernel,
        mesh=mesh,
        out_type=jax.ShapeDtypeStruct((NC, N_PAD, D), jnp.float32),
        scratch_types=scratch,
    )
    def spmm(eidx_hbm, evals_hbm, table_hbm, out_hbm, ibuf, vbuf, *rest):
        rows_v = list(rest[0:n_rb])
        rest = rest[n_rb:]
        if local_table:
            w_v = rest[0]
            rest = rest[1:]
        accum = rest[0]
        sems = rest[1:]
        isem = list(sems[0:8])
        gsem = list(sems[8:8 + n_rb])
        ssem = list(sems[8 + n_rb:8 + 2 * n_rb])
        cid = lax.axis_index("c")
        sid = lax.axis_index("s")
        nc = jnp.where(cid == 0, ca, cb)
        c0 = jnp.where(cid == 0, sid * ca, NS * ca + sid * cb)

        def zero_body(i, _):
            for d in range(D // L):
                rows_v[0][i, pl.ds(d * L, L)] = jnp.zeros((L,), jnp.float32)
            return 0
        lax.fori_loop(0, CH, zero_body, 0)
        row0 = sid * rows_per_tile
        for g in range(n_wb):
            pltpu.sync_copy(rows_v[0], accum.at[pl.ds(row0 + g * CH, CH), :])
        if local_table:
            pltpu.sync_copy(table_hbm, w_v)

        plsc.subcore_barrier()

        def issue_idx(g, slot):
            pltpu.async_copy(eidx_hbm.at[c0 + g], ibuf.at[slot], isem[slot])
            pltpu.async_copy(evals_hbm.at[c0 + g], vbuf.at[slot], isem[slot])

        def wait_idx(slot):
            pltpu.make_async_copy(eidx_hbm.at[0], ibuf.at[slot], isem[slot]).wait()
            pltpu.make_async_copy(evals_hbm.at[0], vbuf.at[slot], isem[slot]).wait()

        def wait_rows(buf, sem):
            pltpu.make_async_copy(out_hbm.at[0, pl.ds(0, CH), :], buf, sem).wait()

        def scale_scatter(rb, slot):
            def scale_body(g16, _):
                v16 = vbuf[slot, pl.ds(g16 * L, L)]
                for j in range(L):
                    i = g16 * L + j
                    vsp = jnp.full((L,), v16[j], jnp.float32)
                    for d in range(D // L):
                        rows_v[rb][i, pl.ds(d * L, L)] = (
                            rows_v[rb][i, pl.ds(d * L, L)] * vsp)
                return 0
            lax.fori_loop(0, CH // L, scale_body, 0)
            pltpu.async_copy(rows_v[rb], accum.at[ibuf.at[slot, 1]], ssem[rb],
                             add=True)

        def wmul_scatter(rb, slot):
            def scale_body(g16, _):
                v16 = vbuf[slot, pl.ds(g16 * L, L)]
                c16 = ibuf[slot, 0, pl.ds(g16 * L, L)]
                for j in range(L):
                    i = g16 * L + j
                    vsp = jnp.full((L,), v16[j], jnp.float32)
                    col = c16[j]
                    for d in range(D // L):
                        rows_v[rb][i, pl.ds(d * L, L)] = (
                            w_v[col, pl.ds(d * L, L)] * vsp)
                return 0
            lax.fori_loop(0, CH // L, scale_body, 0)
            pltpu.async_copy(rows_v[rb], accum.at[ibuf.at[slot, 1]], ssem[rb],
                             add=True)

        issue_idx(0, 0)
        issue_idx(1, 1)

        def pipe_body(it, _):
            for k in range(8):
                g = it * 8 + k
                rb = k % n_rb
                rbp = (k + n_rb - DEPTH) % n_rb
                slot = k % 8
                slotp = (k + 8 - DEPTH) % 8
                slotn = (k + 2) % 8

                @pl.when(jnp.logical_and(g >= n_rb, g < nc + n_rb))
                def _():
                    wait_rows(rows_v[rb], ssem[rb])

                if not local_table:
                    @pl.when(g < nc)
                    def _():
                        wait_idx(slot)
                        pltpu.async_copy(table_hbm.at[ibuf.at[slot, 0]],
                                         rows_v[rb], gsem[rb])

                    @pl.when(jnp.logical_and(g >= DEPTH, g < nc + DEPTH))
                    def _():
                        wait_rows(rows_v[rbp], gsem[rbp])
                        scale_scatter(rbp, slotp)
                else:
                    @pl.when(g < nc)
                    def _():
                        wait_idx(slot)
                        wmul_scatter(rb, slot)

                @pl.when(g + 2 < nc)
                def _():
                    issue_idx(g + 2, slotn)
            return 0
        lax.fori_loop(0, nc // 8 + 1, pipe_body, 0)

        plsc.subcore_barrier()

        for g in range(n_wb):
            r = row0 + g * CH
            pltpu.sync_copy(accum.at[pl.ds(r, CH), :], rows_v[0])
            pltpu.sync_copy(rows_v[0], out_hbm.at[cid, pl.ds(r, CH), :])

    return spmm


def _pack_edges(indices, vals, e_pad):
    e = vals.shape[0]
    pad = e_pad - e
    rows = jnp.concatenate([indices[0].astype(jnp.int32), jnp.zeros((pad,), jnp.int32)])
    cols = jnp.concatenate([indices[1].astype(jnp.int32), jnp.zeros((pad,), jnp.int32)])
    v = jnp.concatenate([vals.astype(jnp.float32), jnp.zeros((pad,), jnp.float32)])
    return (jnp.stack([cols.reshape(-1, CH), rows.reshape(-1, CH)], axis=1),
            v.reshape(-1, CH))


def kernel(phi_indices, phi_values, phi_inverse_indices, phi_inverse_values,
           feature_indices, feature_values, weight_matrix, diagonal_weight_filter,
           dropout):
    f32 = jnp.float32
    w = weight_matrix.astype(f32)
    theta = diagonal_weight_filter.reshape(-1).astype(f32)
    theta_pad = jnp.concatenate([theta, jnp.zeros((N_PAD - N,), f32)])

    grain = 32 * CH * 8
    e_feat = grain * -(-feature_values.shape[0] // grain)
    e_phi = grain * -(-phi_values.shape[0] // grain)

    feat_i, feat_v = _pack_edges(feature_indices, feature_values, e_feat)
    pinv_i, pinv_v = _pack_edges(phi_inverse_indices, phi_inverse_values, e_phi)
    phi_i, phi_v = _pack_edges(phi_indices, phi_values, e_phi)

    nphi = e_phi // (NC * NS) // CH
    spmm_w = _make_spmm(e_feat, local_table=True)
    spmm_n = _make_spmm(e_phi, local_table=False,
                        split=(nphi - nphi // 2, nphi + nphi // 2))

    p_a = spmm_w(feat_i, feat_v, w)
    filtered = p_a[0] + p_a[1]
    p_b = spmm_n(pinv_i, pinv_v, filtered)
    tmp_scaled = theta_pad[:, None] * (p_b[0] + p_b[1])
    p_c = spmm_n(phi_i, phi_v, tmp_scaled)
    out = jax.nn.relu(p_c[0] + p_c[1])
    return out[:N].reshape(N, 1, D)


# speedup vs baseline: 1.0873x; 1.0873x over previous
import functools

import jax
import jax.numpy as jnp
from jax import lax
from jax.experimental import pallas as pl
from jax.experimental.pallas import tpu as pltpu
from jax.experimental.pallas import tpu_sc as plsc

N = 10000
N_PAD = 10240
D = 128
NC = 2
NS = 16
L = 16
CH = 64


def _make_spmm(e_pad, local_table, split=None):
    per_tile = e_pad // (NC * NS)
    n_chunks = per_tile // CH
    assert n_chunks * CH == per_tile and n_chunks % 8 == 0
    ca, cb = split if split else (n_chunks, n_chunks)
    assert ca + cb == 2 * n_chunks and ca % 8 == 0 and cb % 8 == 0
    rows_per_tile = N_PAD // NS
    n_wb = rows_per_tile // CH
    n_rb = 2 if local_table else 4
    DEPTH = 1 if local_table else 3

    mesh = plsc.VectorSubcoreMesh(core_axis_name="c", subcore_axis_name="s")

    scratch = [
        pltpu.VMEM((8, 2, CH), jnp.int32),
        pltpu.VMEM((8, CH), jnp.float32),
    ]
    scratch += [pltpu.VMEM((CH, D), jnp.float32)] * n_rb
    if local_table:
        scratch.append(pltpu.VMEM((D, D), jnp.float32))
    scratch.append(pltpu.VMEM_SHARED((N_PAD, D), jnp.float32))
    scratch += [pltpu.SemaphoreType.DMA] * (8 + 2 * n_rb)

    @functools.partial(
        pl.kernel,
        mesh=mesh,
        out_type=jax.ShapeDtypeStruct((NC, N_PAD, D), jnp.float32),
        scratch_types=scratch,
    )
    def spmm(eidx_hbm, evals_hbm, table_hbm, out_hbm, ibuf, vbuf, *rest):
        rows_v = list(rest[0:n_rb])
        rest = rest[n_rb:]
        if local_table:
            w_v = rest[0]
            rest = rest[1:]
        accum = rest[0]
        sems = rest[1:]
        isem = list(sems[0:8])
        gsem = list(sems[8:8 + n_rb])
        ssem = list(sems[8 + n_rb:8 + 2 * n_rb])
        cid = lax.axis_index("c")
        sid = lax.axis_index("s")
        nc = jnp.where(cid == 0, ca, cb)
        c0 = jnp.where(cid == 0, sid * ca, NS * ca + sid * cb)

        def zero_body(i, _):
            for d in range(D // L):
                rows_v[0][i, pl.ds(d * L, L)] = jnp.zeros((L,), jnp.float32)
            return 0
        lax.fori_loop(0, CH, zero_body, 0)
        row0 = sid * rows_per_tile
        for g in range(n_wb):
            pltpu.sync_copy(rows_v[0], accum.at[pl.ds(row0 + g * CH, CH), :])
        if local_table:
            pltpu.sync_copy(table_hbm, w_v)

        plsc.subcore_barrier()

        def issue_idx(g, slot):
            pltpu.async_copy(eidx_hbm.at[c0 + g], ibuf.at[slot], isem[slot])
            pltpu.async_copy(evals_hbm.at[c0 + g], vbuf.at[slot], isem[slot])

        def wait_idx(slot):
            pltpu.make_async_copy(eidx_hbm.at[0], ibuf.at[slot], isem[slot]).wait()
            pltpu.make_async_copy(evals_hbm.at[0], vbuf.at[slot], isem[slot]).wait()

        def wait_rows(buf, sem):
            pltpu.make_async_copy(out_hbm.at[0, pl.ds(0, CH), :], buf, sem).wait()

        def scale_scatter(rb, slot):
            def scale_body(g16, _):
                v16 = vbuf[slot, pl.ds(g16 * L, L)]
                for j in range(L):
                    i = g16 * L + j
                    vsp = jnp.full((L,), v16[j], jnp.float32)
                    for d in range(D // L):
                        rows_v[rb][i, pl.ds(d * L, L)] = (
                            rows_v[rb][i, pl.ds(d * L, L)] * vsp)
                return 0
            lax.fori_loop(0, CH // L, scale_body, 0)
            pltpu.async_copy(rows_v[rb], accum.at[ibuf.at[slot, 1]], ssem[rb],
                             add=True)

        def wmul_scatter(rb, slot):
            def scale_body(g16, _):
                v16 = vbuf[slot, pl.ds(g16 * L, L)]
                c16 = ibuf[slot, 0, pl.ds(g16 * L, L)]
                for j in range(L):
                    i = g16 * L + j
                    vsp = jnp.full((L,), v16[j], jnp.float32)
                    col = c16[j]
                    for d in range(D // L):
                        rows_v[rb][i, pl.ds(d * L, L)] = (
                            w_v[col, pl.ds(d * L, L)] * vsp)
                return 0
            lax.fori_loop(0, CH // L, scale_body, 0)
            pltpu.async_copy(rows_v[rb], accum.at[ibuf.at[slot, 1]], ssem[rb],
                             add=True)

        issue_idx(0, 0)
        issue_idx(1, 1)

        def pipe_body(it, _):
            for k in range(8):
                g = it * 8 + k
                rb = k % n_rb
                rbp = (k + n_rb - DEPTH) % n_rb
                slot = k % 8
                slotp = (k + 8 - DEPTH) % 8
                slotn = (k + 2) % 8

                @pl.when(jnp.logical_and(g >= n_rb, g < nc + n_rb))
                def _():
                    wait_rows(rows_v[rb], ssem[rb])

                if not local_table:
                    @pl.when(g < nc)
                    def _():
                        wait_idx(slot)
                        pltpu.async_copy(table_hbm.at[ibuf.at[slot, 0]],
                                         rows_v[rb], gsem[rb])

                    @pl.when(jnp.logical_and(g >= DEPTH, g < nc + DEPTH))
                    def _():
                        wait_rows(rows_v[rbp], gsem[rbp])
                        scale_scatter(rbp, slotp)
                else:
                    @pl.when(g < nc)
                    def _():
                        wait_idx(slot)
                        wmul_scatter(rb, slot)

                @pl.when(g + 2 < nc)
                def _():
                    issue_idx(g + 2, slotn)
            return 0
        lax.fori_loop(0, nc // 8 + 1, pipe_body, 0)

        plsc.subcore_barrier()

        for g in range(n_wb):
            r = row0 + g * CH
            pltpu.sync_copy(accum.at[pl.ds(r, CH), :], rows_v[0])
            pltpu.sync_copy(rows_v[0], out_hbm.at[cid, pl.ds(r, CH), :])

    return spmm


def _pack_edges(indices, vals, e_pad):
    e = vals.shape[0]
    pad = e_pad - e
    rows = jnp.concatenate([indices[0].astype(jnp.int32), jnp.zeros((pad,), jnp.int32)])
    cols = jnp.concatenate([indices[1].astype(jnp.int32), jnp.zeros((pad,), jnp.int32)])
    v = jnp.concatenate([vals.astype(jnp.float32), jnp.zeros((pad,), jnp.float32)])
    return (jnp.stack([cols.reshape(-1, CH), rows.reshape(-1, CH)], axis=1),
            v.reshape(-1, CH))


def kernel(phi_indices, phi_values, phi_inverse_indices, phi_inverse_values,
           feature_indices, feature_values, weight_matrix, diagonal_weight_filter,
           dropout):
    f32 = jnp.float32
    w = weight_matrix.astype(f32)
    theta = diagonal_weight_filter.reshape(-1).astype(f32)
    theta_pad = jnp.concatenate([theta, jnp.zeros((N_PAD - N,), f32)])

    grain = 32 * CH * 8
    e_feat = grain * -(-feature_values.shape[0] // grain)
    e_phi = grain * -(-phi_values.shape[0] // grain)

    feat_i, feat_v = _pack_edges(feature_indices, feature_values, e_feat)
    pinv_i, pinv_v = _pack_edges(phi_inverse_indices, phi_inverse_values, e_phi)
    phi_i, phi_v = _pack_edges(phi_indices, phi_values, e_phi)

    nphi = e_phi // (NC * NS) // CH
    spmm_w = _make_spmm(e_feat, local_table=True)
    spmm_n = _make_spmm(e_phi, local_table=False,
                        split=(nphi + nphi // 4, nphi - nphi // 4))

    p_a = spmm_w(feat_i, feat_v, w)
    filtered = p_a[0] + p_a[1]
    p_b = spmm_n(pinv_i, pinv_v, filtered)
    tmp_scaled = theta_pad[:, None] * (p_b[0] + p_b[1])
    p_c = spmm_n(phi_i, phi_v, tmp_scaled)
    out = jax.nn.relu(p_c[0] + p_c[1])
    return out[:N].reshape(N, 1, D)


# speedup vs baseline: 1.1726x; 1.0784x over previous
import functools

import jax
import jax.numpy as jnp
from jax import lax
from jax.experimental import pallas as pl
from jax.experimental.pallas import tpu as pltpu
from jax.experimental.pallas import tpu_sc as plsc

N = 10000
N_PAD = 10240
D = 128
NC = 2
NS = 16
L = 16
CH = 64


def _make_spmm(e_pad, local_table, split=None):
    per_tile = e_pad // (NC * NS)
    n_chunks = per_tile // CH
    assert n_chunks * CH == per_tile and n_chunks % 8 == 0
    ca, cb = split if split else (n_chunks, n_chunks)
    assert ca + cb == 2 * n_chunks and ca % 8 == 0 and cb % 8 == 0
    rows_per_tile = N_PAD // NS
    n_wb = rows_per_tile // CH
    n_rb = 2 if local_table else 4
    DEPTH = 1 if local_table else 3

    mesh = plsc.VectorSubcoreMesh(core_axis_name="c", subcore_axis_name="s")

    scratch = [
        pltpu.VMEM((8, 2, CH), jnp.int32),
        pltpu.VMEM((8, CH), jnp.float32),
    ]
    scratch += [pltpu.VMEM((CH, D), jnp.float32)] * n_rb
    if local_table:
        scratch.append(pltpu.VMEM((D, D), jnp.float32))
    scratch.append(pltpu.VMEM_SHARED((N_PAD, D), jnp.float32))
    scratch += [pltpu.SemaphoreType.DMA] * (8 + 2 * n_rb)

    @functools.partial(
        pl.kernel,
        mesh=mesh,
        out_type=jax.ShapeDtypeStruct((NC, N_PAD, D), jnp.float32),
        scratch_types=scratch,
    )
    def spmm(eidx_hbm, evals_hbm, table_hbm, out_hbm, ibuf, vbuf, *rest):
        rows_v = list(rest[0:n_rb])
        rest = rest[n_rb:]
        if local_table:
            w_v = rest[0]
            rest = rest[1:]
        accum = rest[0]
        sems = rest[1:]
        isem = list(sems[0:8])
        gsem = list(sems[8:8 + n_rb])
        ssem = list(sems[8 + n_rb:8 + 2 * n_rb])
        cid = lax.axis_index("c")
        sid = lax.axis_index("s")
        nc = jnp.where(cid == 0, ca, cb)
        c0 = jnp.where(cid == 0, sid * ca, NS * ca + sid * cb)

        def zero_body(i, _):
            for d in range(D // L):
                rows_v[0][i, pl.ds(d * L, L)] = jnp.zeros((L,), jnp.float32)
            return 0
        lax.fori_loop(0, CH, zero_body, 0)
        row0 = sid * rows_per_tile
        for g in range(n_wb):
            pltpu.sync_copy(rows_v[0], accum.at[pl.ds(row0 + g * CH, CH), :])
        if local_table:
            pltpu.sync_copy(table_hbm, w_v)

        plsc.subcore_barrier()

        def issue_idx(g, slot):
            pltpu.async_copy(eidx_hbm.at[c0 + g], ibuf.at[slot], isem[slot])
            pltpu.async_copy(evals_hbm.at[c0 + g], vbuf.at[slot], isem[slot])

        def wait_idx(slot):
            pltpu.make_async_copy(eidx_hbm.at[0], ibuf.at[slot], isem[slot]).wait()
            pltpu.make_async_copy(evals_hbm.at[0], vbuf.at[slot], isem[slot]).wait()

        def wait_rows(buf, sem):
            pltpu.make_async_copy(out_hbm.at[0, pl.ds(0, CH), :], buf, sem).wait()

        def scale_scatter(rb, slot):
            def scale_body(g16, _):
                v16 = vbuf[slot, pl.ds(g16 * L, L)]
                for j in range(L):
                    i = g16 * L + j
                    vsp = jnp.full((L,), v16[j], jnp.float32)
                    for d in range(D // L):
                        rows_v[rb][i, pl.ds(d * L, L)] = (
                            rows_v[rb][i, pl.ds(d * L, L)] * vsp)
                return 0
            lax.fori_loop(0, CH // L, scale_body, 0)
            pltpu.async_copy(rows_v[rb], accum.at[ibuf.at[slot, 1]], ssem[rb],
                             add=True)

        def wmul_scatter(rb, slot):
            def scale_body(g16, _):
                v16 = vbuf[slot, pl.ds(g16 * L, L)]
                c16 = ibuf[slot, 0, pl.ds(g16 * L, L)]
                for j in range(L):
                    i = g16 * L + j
                    vsp = jnp.full((L,), v16[j], jnp.float32)
                    col = c16[j]
                    for d in range(D // L):
                        rows_v[rb][i, pl.ds(d * L, L)] = (
                            w_v[col, pl.ds(d * L, L)] * vsp)
                return 0
            lax.fori_loop(0, CH // L, scale_body, 0)
            pltpu.async_copy(rows_v[rb], accum.at[ibuf.at[slot, 1]], ssem[rb],
                             add=True)

        issue_idx(0, 0)
        issue_idx(1, 1)

        def pipe_body(it, _):
            for k in range(8):
                g = it * 8 + k
                rb = k % n_rb
                rbp = (k + n_rb - DEPTH) % n_rb
                slot = k % 8
                slotp = (k + 8 - DEPTH) % 8
                slotn = (k + 2) % 8

                @pl.when(jnp.logical_and(g >= n_rb, g < nc + n_rb))
                def _():
                    wait_rows(rows_v[rb], ssem[rb])

                if not local_table:
                    @pl.when(g < nc)
                    def _():
                        wait_idx(slot)
                        pltpu.async_copy(table_hbm.at[ibuf.at[slot, 0]],
                                         rows_v[rb], gsem[rb])

                    @pl.when(jnp.logical_and(g >= DEPTH, g < nc + DEPTH))
                    def _():
                        wait_rows(rows_v[rbp], gsem[rbp])
                        scale_scatter(rbp, slotp)
                else:
                    @pl.when(g < nc)
                    def _():
                        wait_idx(slot)
                        wmul_scatter(rb, slot)

                @pl.when(g + 2 < nc)
                def _():
                    issue_idx(g + 2, slotn)
            return 0
        lax.fori_loop(0, nc // 8 + 1, pipe_body, 0)

        plsc.subcore_barrier()

        for g in range(n_wb):
            r = row0 + g * CH
            pltpu.sync_copy(accum.at[pl.ds(r, CH), :], rows_v[0])
            pltpu.sync_copy(rows_v[0], out_hbm.at[cid, pl.ds(r, CH), :])

    return spmm


def _pack_edges(indices, vals, e_pad):
    e = vals.shape[0]
    pad = e_pad - e
    rows = jnp.concatenate([indices[0].astype(jnp.int32), jnp.zeros((pad,), jnp.int32)])
    cols = jnp.concatenate([indices[1].astype(jnp.int32), jnp.zeros((pad,), jnp.int32)])
    v = jnp.concatenate([vals.astype(jnp.float32), jnp.zeros((pad,), jnp.float32)])
    return (jnp.stack([cols.reshape(-1, CH), rows.reshape(-1, CH)], axis=1),
            v.reshape(-1, CH))


def kernel(phi_indices, phi_values, phi_inverse_indices, phi_inverse_values,
           feature_indices, feature_values, weight_matrix, diagonal_weight_filter,
           dropout):
    f32 = jnp.float32
    w = weight_matrix.astype(f32)
    theta = diagonal_weight_filter.reshape(-1).astype(f32)
    theta_pad = jnp.concatenate([theta, jnp.zeros((N_PAD - N,), f32)])

    grain = 32 * CH * 8
    e_feat = grain * -(-feature_values.shape[0] // grain)
    e_phi = grain * -(-phi_values.shape[0] // grain)

    feat_i, feat_v = _pack_edges(feature_indices, feature_values, e_feat)
    pinv_i, pinv_v = _pack_edges(phi_inverse_indices, phi_inverse_values, e_phi)
    phi_i, phi_v = _pack_edges(phi_indices, phi_values, e_phi)

    nphi = e_phi // (NC * NS) // CH
    spmm_w = _make_spmm(e_feat, local_table=True)
    spmm_n = _make_spmm(e_phi, local_table=False,
                        split=(nphi + nphi * 3 // 4, nphi - nphi * 3 // 4))

    p_a = spmm_w(feat_i, feat_v, w)
    filtered = p_a[0] + p_a[1]
    p_b = spmm_n(pinv_i, pinv_v, filtered)
    tmp_scaled = theta_pad[:, None] * (p_b[0] + p_b[1])
    p_c = spmm_n(phi_i, phi_v, tmp_scaled)
    out = jax.nn.relu(p_c[0] + p_c[1])
    return out[:N].reshape(N, 1, D)


# speedup vs baseline: 1.2395x; 1.0571x over previous
import functools

import jax
import jax.numpy as jnp
from jax import lax
from jax.experimental import pallas as pl
from jax.experimental.pallas import tpu as pltpu
from jax.experimental.pallas import tpu_sc as plsc

N = 10000
N_PAD = 10240
D = 128
NC = 2
NS = 16
L = 16
CH = 64


def _make_spmm(e_pad, local_table, split=None):
    per_tile = e_pad // (NC * NS)
    n_chunks = per_tile // CH
    assert n_chunks * CH == per_tile and n_chunks % 8 == 0
    ca, cb = split if split else (n_chunks, n_chunks)
    assert ca + cb == 2 * n_chunks and ca % 8 == 0 and cb % 8 == 0
    rows_per_tile = N_PAD // NS
    n_wb = rows_per_tile // CH
    n_rb = 2 if local_table else 4
    DEPTH = 1 if local_table else 3

    mesh = plsc.VectorSubcoreMesh(core_axis_name="c", subcore_axis_name="s")

    scratch = [
        pltpu.VMEM((8, 2, CH), jnp.int32),
        pltpu.VMEM((8, CH), jnp.float32),
    ]
    scratch += [pltpu.VMEM((CH, D), jnp.float32)] * n_rb
    if local_table:
        scratch.append(pltpu.VMEM((D, D), jnp.float32))
    scratch.append(pltpu.VMEM_SHARED((N_PAD, D), jnp.float32))
    scratch += [pltpu.SemaphoreType.DMA] * (8 + 2 * n_rb)

    @functools.partial(
        pl.kernel,
        mesh=mesh,
        out_type=jax.ShapeDtypeStruct((NC, N_PAD, D), jnp.float32),
        scratch_types=scratch,
    )
    def spmm(eidx_hbm, evals_hbm, table_hbm, out_hbm, ibuf, vbuf, *rest):
        rows_v = list(rest[0:n_rb])
        rest = rest[n_rb:]
        if local_table:
            w_v = rest[0]
            rest = rest[1:]
        accum = rest[0]
        sems = rest[1:]
        isem = list(sems[0:8])
        gsem = list(sems[8:8 + n_rb])
        ssem = list(sems[8 + n_rb:8 + 2 * n_rb])
        cid = lax.axis_index("c")
        sid = lax.axis_index("s")
        nc = jnp.where(cid == 0, ca, cb)
        c0 = jnp.where(cid == 0, sid * ca, NS * ca + sid * cb)

        def zero_body(i, _):
            for d in range(D // L):
                rows_v[0][i, pl.ds(d * L, L)] = jnp.zeros((L,), jnp.float32)
            return 0
        lax.fori_loop(0, CH, zero_body, 0)
        row0 = sid * rows_per_tile
        for g in range(n_wb):
            pltpu.sync_copy(rows_v[0], accum.at[pl.ds(row0 + g * CH, CH), :])
        if local_table:
            pltpu.sync_copy(table_hbm, w_v)

        plsc.subcore_barrier()

        def issue_idx(g, slot):
            pltpu.async_copy(eidx_hbm.at[c0 + g], ibuf.at[slot], isem[slot])
            pltpu.async_copy(evals_hbm.at[c0 + g], vbuf.at[slot], isem[slot])

        def wait_idx(slot):
            pltpu.make_async_copy(eidx_hbm.at[0], ibuf.at[slot], isem[slot]).wait()
            pltpu.make_async_copy(evals_hbm.at[0], vbuf.at[slot], isem[slot]).wait()

        def wait_rows(buf, sem):
            pltpu.make_async_copy(out_hbm.at[0, pl.ds(0, CH), :], buf, sem).wait()

        def scale_scatter(rb, slot):
            def scale_body(g16, _):
                v16 = vbuf[slot, pl.ds(g16 * L, L)]
                for j in range(L):
                    i = g16 * L + j
                    vsp = jnp.full((L,), v16[j], jnp.float32)
                    for d in range(D // L):
                        rows_v[rb][i, pl.ds(d * L, L)] = (
                            rows_v[rb][i, pl.ds(d * L, L)] * vsp)
                return 0
            lax.fori_loop(0, CH // L, scale_body, 0)
            pltpu.async_copy(rows_v[rb], accum.at[ibuf.at[slot, 1]], ssem[rb],
                             add=True)

        def wmul_scatter(rb, slot):
            def scale_body(g16, _):
                v16 = vbuf[slot, pl.ds(g16 * L, L)]
                c16 = ibuf[slot, 0, pl.ds(g16 * L, L)]
                for j in range(L):
                    i = g16 * L + j
                    vsp = jnp.full((L,), v16[j], jnp.float32)
                    col = c16[j]
                    for d in range(D // L):
                        rows_v[rb][i, pl.ds(d * L, L)] = (
                            w_v[col, pl.ds(d * L, L)] * vsp)
                return 0
            lax.fori_loop(0, CH // L, scale_body, 0)
            pltpu.async_copy(rows_v[rb], accum.at[ibuf.at[slot, 1]], ssem[rb],
                             add=True)

        issue_idx(0, 0)
        issue_idx(1, 1)

        def pipe_body(it, _):
            for k in range(8):
                g = it * 8 + k
                rb = k % n_rb
                rbp = (k + n_rb - DEPTH) % n_rb
                slot = k % 8
                slotp = (k + 8 - DEPTH) % 8
                slotn = (k + 2) % 8

                @pl.when(jnp.logical_and(g >= n_rb, g < nc + n_rb))
                def _():
                    wait_rows(rows_v[rb], ssem[rb])

                if not local_table:
                    @pl.when(g < nc)
                    def _():
                        wait_idx(slot)
                        pltpu.async_copy(table_hbm.at[ibuf.at[slot, 0]],
                                         rows_v[rb], gsem[rb])

                    @pl.when(jnp.logical_and(g >= DEPTH, g < nc + DEPTH))
                    def _():
                        wait_rows(rows_v[rbp], gsem[rbp])
                        scale_scatter(rbp, slotp)
                else:
                    @pl.when(g < nc)
                    def _():
                        wait_idx(slot)
                        wmul_scatter(rb, slot)

                @pl.when(g + 2 < nc)
                def _():
                    issue_idx(g + 2, slotn)
            return 0
        lax.fori_loop(0, nc // 8 + 1, pipe_body, 0)

        plsc.subcore_barrier()

        for g in range(n_wb):
            r = row0 + g * CH
            pltpu.sync_copy(accum.at[pl.ds(r, CH), :], rows_v[0])
            pltpu.sync_copy(rows_v[0], out_hbm.at[cid, pl.ds(r, CH), :])

    return spmm


def _pack_edges(indices, vals, e_pad):
    e = vals.shape[0]
    pad = e_pad - e
    rows = jnp.concatenate([indices[0].astype(jnp.int32), jnp.zeros((pad,), jnp.int32)])
    cols = jnp.concatenate([indices[1].astype(jnp.int32), jnp.zeros((pad,), jnp.int32)])
    v = jnp.concatenate([vals.astype(jnp.float32), jnp.zeros((pad,), jnp.float32)])
    return (jnp.stack([cols.reshape(-1, CH), rows.reshape(-1, CH)], axis=1),
            v.reshape(-1, CH))


def kernel(phi_indices, phi_values, phi_inverse_indices, phi_inverse_values,
           feature_indices, feature_values, weight_matrix, diagonal_weight_filter,
           dropout):
    f32 = jnp.float32
    w = weight_matrix.astype(f32)
    theta = diagonal_weight_filter.reshape(-1).astype(f32)
    theta_pad = jnp.concatenate([theta, jnp.zeros((N_PAD - N,), f32)])

    grain = 32 * CH * 8
    e_feat = grain * -(-feature_values.shape[0] // grain)
    e_phi = grain * -(-phi_values.shape[0] // grain)

    feat_i, feat_v = _pack_edges(feature_indices, feature_values, e_feat)
    pinv_i, pinv_v = _pack_edges(phi_inverse_indices, phi_inverse_values, e_phi)
    phi_i, phi_v = _pack_edges(phi_indices, phi_values, e_phi)

    nphi = e_phi // (NC * NS) // CH
    spmm_w = _make_spmm(e_feat, local_table=True)
    spmm_n = _make_spmm(e_phi, local_table=False,
                        split=(2 * nphi - 8, 8))

    p_a = spmm_w(feat_i, feat_v, w)
    filtered = p_a[0] + p_a[1]
    p_b = spmm_n(pinv_i, pinv_v, filtered)
    tmp_scaled = theta_pad[:, None] * (p_b[0] + p_b[1])
    p_c = spmm_n(phi_i, phi_v, tmp_scaled)
    out = jax.nn.relu(p_c[0] + p_c[1])
    return out[:N].reshape(N, 1, D)
